# Initial kernel scaffold; baseline (speedup 1.0000x reference)
import jax
import jax.numpy as jnp
from jax.experimental import pallas as pl


def kernel(x, edges, W_enc, bn_gamma, bn_beta, ext_w, source_beta, W_pair, W1, b1, W2, b2):
    raise NotImplementedError("write your pallas kernel here")


# speedup vs baseline: 15.4586x; 15.4586x over previous
import functools

import jax
import jax.numpy as jnp
from jax import lax
from jax.experimental import pallas as pl
from jax.experimental.pallas import tpu as pltpu
from jax.experimental.pallas import tpu_sc as plsc

N = 10000
E = 320000
H = 128
LAYERS = 4
STEP = 0.1

NC = 2
NS = 16
NW = NC * NS
RPS = N // NS
NP1 = 10240
RPS1 = NP1 // NS
NP2 = 10240
RPS2 = NP2 // NS

GS_W = 125
GS_K = 40
GS_NBLK = E // (NW * GS_K * GS_W)
DG_W = 80
DG_ITERS = E // (NW * DG_W)

_f32 = jnp.float32


def _vector_mesh():
    return plsc.VectorSubcoreMesh(core_axis_name="c", subcore_axis_name="s")


@functools.partial(
    pl.kernel,
    out_type=jax.ShapeDtypeStruct((2 * NP1,), _f32),
    mesh=_vector_mesh(),
    scratch_types=[
        pltpu.VMEM((DG_ITERS, DG_W), jnp.int32),
        pltpu.VMEM((DG_W,), _f32),
        pltpu.VMEM((RPS1,), _f32),
        pltpu.VMEM_SHARED((NP1,), _f32),
    ],
)
def _sc_degree(col_hbm, ones_hbm, zeros_hbm, out_hbm, col_v, ones_v, stage_v, deg_sh):
    c = lax.axis_index("c")
    s = lax.axis_index("s")
    wid = c * NS + s
    pltpu.sync_copy(col_hbm.at[wid], col_v)
    pltpu.sync_copy(ones_hbm, ones_v)
    pltpu.sync_copy(zeros_hbm.at[pl.ds(s * RPS1, RPS1)], stage_v)
    pltpu.sync_copy(stage_v, deg_sh.at[pl.ds(s * RPS1, RPS1)])
    plsc.subcore_barrier()

    @pl.loop(0, DG_ITERS)
    def _(j):
        pltpu.sync_copy(ones_v, deg_sh.at[col_v.at[j]], add=True)

    plsc.subcore_barrier()
    pltpu.sync_copy(deg_sh.at[pl.ds(s * RPS1, RPS1)], stage_v)
    pltpu.sync_copy(stage_v, out_hbm.at[pl.ds(c * NP1 + s * RPS1, RPS1)])


@functools.partial(
    pl.kernel,
    out_type=jax.ShapeDtypeStruct((2 * NP2, H), _f32),
    mesh=_vector_mesh(),
    scratch_types=[
        pltpu.VMEM((GS_K, 2, GS_W), jnp.int32),
        pltpu.VMEM((GS_W, H), _f32),
        pltpu.VMEM((GS_W, H), _f32),
        pltpu.VMEM_SHARED((NP2, H), _f32),
        pltpu.SemaphoreType.DMA,
        pltpu.SemaphoreType.DMA,
    ],
)
def _sc_gather_scatter(outp_hbm, rc_hbm, zeros_hbm, out_hbm,
                       rcb, buf0, buf1, agg_sh, sem0, sem1):
    c = lax.axis_index("c")
    s = lax.axis_index("s")
    wid = c * NS + s
    pltpu.sync_copy(zeros_hbm.at[pl.ds(s * RPS2, RPS2)], agg_sh.at[pl.ds(s * RPS2, RPS2)])
    plsc.subcore_barrier()

    @pl.loop(0, GS_NBLK)
    def _(b):
        pltpu.sync_copy(rc_hbm.at[wid, b], rcb)
        @pl.loop(0, GS_K, step=2)
        def _(k):
            cp0 = pltpu.async_copy(outp_hbm.at[rcb.at[k, 0]], buf0, sem0)
            cp1 = pltpu.async_copy(outp_hbm.at[rcb.at[k + 1, 0]], buf1, sem1)
            cp0.wait()
            pltpu.sync_copy(buf0, agg_sh.at[rcb.at[k, 1]], add=True)
            cp1.wait()
            pltpu.sync_copy(buf1, agg_sh.at[rcb.at[k + 1, 1]], add=True)

    plsc.subcore_barrier()
    pltpu.sync_copy(agg_sh.at[pl.ds(s * RPS2, RPS2)],
                    out_hbm.at[pl.ds(c * NP2 + s * RPS2, RPS2)])


def _dotT(a, w):
    return lax.dot_general(a, w, (((1,), (1,)), ((), ())),
                           preferred_element_type=_f32)


def _elu(g):
    return jnp.where(g > 0, g, jnp.exp(jnp.minimum(g, 0.0)) - 1.0)


def _enc_body(x_ref, wenc_ref, g_ref, b_ref, wpair_ref, h0_ref, weff_ref):
    h = _dotT(x_ref[...], wenc_ref[...])
    mu = jnp.sum(h, axis=0, keepdims=True) / N
    var = jnp.sum(h * h, axis=0, keepdims=True) / N - mu * mu
    h0_ref[...] = (h - mu) * (g_ref[...] / jnp.sqrt(var + 1e-5)) + b_ref[...]

    wp = wpair_ref[...]
    w0raw = wp[:, :H]
    i2 = lax.broadcasted_iota(jnp.int32, (H, H), 0)
    j2 = lax.broadcasted_iota(jnp.int32, (H, H), 1)
    w0 = jnp.where(j2 > i2, w0raw, 0.0)
    w0 = w0 + w0.T
    q = wp[:, H:H + 1]
    r = wp[:, H + 1:H + 2]
    dvec = q * jnp.sum(jnp.abs(w0), axis=1, keepdims=True) + r
    weff_ref[...] = w0 + jnp.where(i2 == j2, dvec, 0.0)


def _prep_body(degp_ref, h0_ref, weff_ref, deginv_ref, outp_ref):
    deg = degp_ref[:, 0:1] + degp_ref[:, 1:2]
    deginv = jnp.where(deg > 0, lax.rsqrt(jnp.maximum(deg, 1.0)), 0.0)
    deginv_ref[...] = deginv
    outp_ref[...] = _dotT(h0_ref[...], weff_ref[...]) * deginv


def _update(aggp_ref, h_ref, h0_ref, deginv_ref, extw_ref, beta_ref):
    agg = aggp_ref[0, :N] + aggp_ref[1, :N]
    h = h_ref[...]
    g = deginv_ref[...] * agg - h * extw_ref[...] - h0_ref[...] * beta_ref[...]
    return h + STEP * _elu(g)


def _mid_body(aggp_ref, h_ref, h0_ref, deginv_ref, weff_ref, extw_ref,
              beta_ref, hn_ref, outp_ref):
    hn = _update(aggp_ref, h_ref, h0_ref, deginv_ref, extw_ref, beta_ref)
    hn_ref[...] = hn
    outp_ref[...] = _dotT(hn, weff_ref[...]) * deginv_ref[...]


def _fin_body(aggp_ref, h_ref, h0_ref, deginv_ref, extw_ref, beta_ref,
              w1_ref, b1_ref, w2_ref, b2_ref, out_ref):
    hn = _update(aggp_ref, h_ref, h0_ref, deginv_ref, extw_ref, beta_ref)
    t = _elu(_dotT(hn, w1_ref[...]) + b1_ref[...])
    out_ref[...] = _dotT(t, w2_ref[...]) + b2_ref[...]


_enc_call = pl.pallas_call(
    _enc_body,
    out_shape=[jax.ShapeDtypeStruct((N, H), _f32),
               jax.ShapeDtypeStruct((H, H), _f32)],
)
_prep_call = pl.pallas_call(
    _prep_body,
    out_shape=[jax.ShapeDtypeStruct((N, 1), _f32),
               jax.ShapeDtypeStruct((N, H), _f32)],
)
_mid_call = pl.pallas_call(
    _mid_body,
    out_shape=[jax.ShapeDtypeStruct((N, H), _f32),
               jax.ShapeDtypeStruct((N, H), _f32)],
)
_fin_call = pl.pallas_call(
    _fin_body,
    out_shape=jax.ShapeDtypeStruct((N, H), _f32),
)


def kernel(x, edges, W_enc, bn_gamma, bn_beta, ext_w, source_beta, W_pair,
           W1, b1, W2, b2):
    row, col = edges[0], edges[1]
    row4 = row.reshape(NW, GS_NBLK, GS_K, GS_W)
    col4 = col.reshape(NW, GS_NBLK, GS_K, GS_W)
    rc = jnp.stack([row4, col4], axis=3)
    col_dg = col.reshape(NW, DG_ITERS, DG_W)

    zeros2 = jnp.zeros((NP2, H), _f32)
    zeros1 = jnp.zeros((NP1,), _f32)
    ones1 = jnp.ones((DG_W,), _f32)

    deg_flat = _sc_degree(col_dg, ones1, zeros1)
    degp = deg_flat.reshape(2, NP1)[:, :N].T

    g2 = bn_gamma.reshape(1, H)
    b2_ = bn_beta.reshape(1, H)
    h0, weff = _enc_call(x, W_enc, g2, b2_, W_pair)
    deginv, outp = _prep_call(degp, h0, weff)

    extw = ext_w.reshape(1, H)
    sbeta = source_beta.reshape(1, 1)
    h = h0
    out = None
    for layer in range(LAYERS):
        agg_flat = _sc_gather_scatter(outp, rc, zeros2)
        aggp = agg_flat.reshape(2, NP2, H)
        if layer < LAYERS - 1:
            h, outp = _mid_call(aggp, h, h0, deginv, weff, extw, sbeta)
        else:
            out = _fin_call(aggp, h, h0, deginv, extw, sbeta,
                            W1, b1.reshape(1, H), W2, b2.reshape(1, H))
    return out
